# baseline (speedup 1.0000x reference)
import functools

import jax
import jax.numpy as jnp
from jax import lax
from jax.experimental import pallas as pl
from jax.experimental.pallas import tpu as pltpu
from jax.experimental.pallas import tpu_sc as plsc

B, L, D = 16384, 200, 16

L_TC = 136
L_SC = L - L_TC

NC, NS = 2, 16
NW = NC * NS
BW_ = B // NW
Q = 256
NQ = BW_ // Q
CL = 8
NCH = L_SC // CL
RPC = CL * D

LB = 2048
CLt = 8
NBL = B // LB
NLS = L_TC // CLt


@functools.partial(
    pl.kernel,
    mesh=plsc.VectorSubcoreMesh(core_axis_name="c", subcore_axis_name="s"),
    out_type=jax.ShapeDtypeStruct((D, B), jnp.float32),
    scratch_types=[
        pltpu.VMEM((2, RPC, Q), jnp.float32),
        pltpu.VMEM((2, CL, Q), jnp.float32),
        pltpu.VMEM((D, Q), jnp.float32),
        pltpu.SemaphoreType.DMA,
        pltpu.SemaphoreType.DMA,
        pltpu.SemaphoreType.DMA,
        pltpu.SemaphoreType.DMA,
    ],
    compiler_params=pltpu.CompilerParams(use_tc_tiling_on_sc=True),
)
def _agg_sc(x_hbm, m_hbm, out_hbm, xbuf, mbuf, acc, sx0, sx1, sm0, sm1):
    wid = lax.axis_index("c") * NS + lax.axis_index("s")
    sxs = (sx0, sx1)
    sms = (sm0, sm1)

    def x_copy(chunk, lane0, slot):
        return pltpu.make_async_copy(
            x_hbm.at[pl.ds((L_TC // CL + chunk) * RPC, RPC), pl.ds(lane0, Q)],
            xbuf.at[slot], sxs[slot])

    def m_copy(chunk, lane0, slot):
        return pltpu.make_async_copy(
            m_hbm.at[pl.ds(chunk * CL, CL), pl.ds(lane0, Q)],
            mbuf.at[slot], sms[slot])

    def start(chunk, lane0, slot):
        x_copy(chunk, lane0, slot).start()
        m_copy(chunk, lane0, slot).start()

    def compute(slot):
        def blk_body(blk, _):
            o = blk * 16
            mvs = [mbuf[slot, l, pl.ds(o, 16)] for l in range(CL)]
            for d in range(D):
                p = xbuf[slot, d, pl.ds(o, 16)] * mvs[0]
                for l in range(1, CL):
                    p = p + xbuf[slot, l * D + d, pl.ds(o, 16)] * mvs[l]
                plsc.addupdate(acc.at[d, pl.ds(o, 16)], p)
            return 0

        lax.fori_loop(0, Q // 16, blk_body, 0)

    def q_body(q, _):
        lane0 = wid * BW_ + q * Q

        def z_body(r, _):
            for blk in range(Q // 16):
                acc[r, pl.ds(blk * 16, 16)] = jnp.zeros((16,), jnp.float32)
            return 0

        lax.fori_loop(0, D, z_body, 0)
        start(0, lane0, 0)

        def c2_body(c2, _):
            for par in range(2):
                chunk = c2 * 2 + par
                x_copy(chunk, lane0, par).wait()
                m_copy(chunk, lane0, par).wait()
                start(chunk + 1, lane0, 1 - par)
                compute(par)
            return 0

        lax.fori_loop(0, (NCH - 1) // 2, c2_body, 0)
        if NCH % 2 == 0:
            ca = NCH - 2
            x_copy(ca, lane0, ca % 2).wait()
            m_copy(ca, lane0, ca % 2).wait()
            start(ca + 1, lane0, (ca + 1) % 2)
            compute(ca % 2)
        cb = NCH - 1
        x_copy(cb, lane0, cb % 2).wait()
        m_copy(cb, lane0, cb % 2).wait()
        compute(cb % 2)
        pltpu.sync_copy(acc, out_hbm.at[pl.ds(0, D), pl.ds(lane0, Q)])
        return 0

    lax.fori_loop(0, NQ, q_body, 0)


def _tc_body(m_ref, x_ref, o_ref):
    il = pl.program_id(1)
    m = m_ref[...].astype(jnp.float32)
    part = jnp.sum(x_ref[...] * m[:, None, :], axis=0)

    @pl.when(il == 0)
    def _():
        o_ref[...] = part

    @pl.when(il > 0)
    def _():
        o_ref[...] += part


_tc_call = pl.pallas_call(
    _tc_body,
    grid=(NBL, NLS),
    in_specs=[
        pl.BlockSpec((CLt, LB), lambda ib, il: (il, ib)),
        pl.BlockSpec((CLt, D, LB), lambda ib, il: (il, 0, ib)),
    ],
    out_specs=pl.BlockSpec((D, LB), lambda ib, il: (0, ib)),
    out_shape=jax.ShapeDtypeStruct((D, B), jnp.float32),
    compiler_params=pltpu.CompilerParams(
        dimension_semantics=("parallel", "arbitrary")),
)


def kernel(inputs, mask):
    x3 = jnp.transpose(inputs, (1, 2, 0))
    x2 = x3.reshape(L * D, B)
    mt = jnp.transpose(mask, (1, 0))
    m_sc = mt[L_TC:, :].astype(jnp.float32)
    out_sc = _agg_sc(x2, m_sc)
    out_tc = _tc_call(mt.astype(jnp.int8), x3)
    return jnp.transpose(out_sc + out_tc, (1, 0))


# speedup vs baseline: 1.4646x; 1.4646x over previous
import functools

import jax
import jax.numpy as jnp
from jax import lax
from jax.experimental import pallas as pl
from jax.experimental.pallas import tpu as pltpu
from jax.experimental.pallas import tpu_sc as plsc

B, L, D = 16384, 200, 16

L_TC = 128
L_SC = L - L_TC

NC, NS = 2, 16
NW = NC * NS
BW_ = B // NW
Q = 256
NQ = BW_ // Q
CL = 8
NCH = L_SC // CL
RPC = CL * D

LB = 2048
CLt = 32
NBL = B // LB
NLS = L_TC // CLt


@functools.partial(
    pl.kernel,
    mesh=plsc.VectorSubcoreMesh(core_axis_name="c", subcore_axis_name="s"),
    out_type=jax.ShapeDtypeStruct((D, B), jnp.float32),
    scratch_types=[
        pltpu.VMEM((2, RPC, Q), jnp.float32),
        pltpu.VMEM((2, CL, Q), jnp.float32),
        pltpu.VMEM((D, Q), jnp.float32),
        pltpu.SemaphoreType.DMA,
        pltpu.SemaphoreType.DMA,
        pltpu.SemaphoreType.DMA,
        pltpu.SemaphoreType.DMA,
    ],
    compiler_params=pltpu.CompilerParams(use_tc_tiling_on_sc=True),
)
def _agg_sc(x_hbm, m_hbm, out_hbm, xbuf, mbuf, acc, sx0, sx1, sm0, sm1):
    wid = lax.axis_index("c") * NS + lax.axis_index("s")
    sxs = (sx0, sx1)
    sms = (sm0, sm1)

    def x_copy(chunk, lane0, slot):
        return pltpu.make_async_copy(
            x_hbm.at[pl.ds((L_TC // CL + chunk) * RPC, RPC), pl.ds(lane0, Q)],
            xbuf.at[slot], sxs[slot])

    def m_copy(chunk, lane0, slot):
        return pltpu.make_async_copy(
            m_hbm.at[pl.ds(chunk * CL, CL), pl.ds(lane0, Q)],
            mbuf.at[slot], sms[slot])

    def start(chunk, lane0, slot):
        x_copy(chunk, lane0, slot).start()
        m_copy(chunk, lane0, slot).start()

    def compute(slot):
        def blk_body(blk, _):
            o = blk * 16
            mvs = [mbuf[slot, l, pl.ds(o, 16)] for l in range(CL)]
            for d in range(D):
                p = xbuf[slot, d, pl.ds(o, 16)] * mvs[0]
                for l in range(1, CL):
                    p = p + xbuf[slot, l * D + d, pl.ds(o, 16)] * mvs[l]
                plsc.addupdate(acc.at[d, pl.ds(o, 16)], p)
            return 0

        lax.fori_loop(0, Q // 16, blk_body, 0)

    def q_body(q, _):
        lane0 = wid * BW_ + q * Q

        def z_body(r, _):
            for blk in range(Q // 16):
                acc[r, pl.ds(blk * 16, 16)] = jnp.zeros((16,), jnp.float32)
            return 0

        lax.fori_loop(0, D, z_body, 0)
        start(0, lane0, 0)

        def c2_body(c2, _):
            for par in range(2):
                chunk = c2 * 2 + par
                x_copy(chunk, lane0, par).wait()
                m_copy(chunk, lane0, par).wait()
                start(chunk + 1, lane0, 1 - par)
                compute(par)
            return 0

        lax.fori_loop(0, (NCH - 1) // 2, c2_body, 0)
        if NCH % 2 == 0:
            ca = NCH - 2
            x_copy(ca, lane0, ca % 2).wait()
            m_copy(ca, lane0, ca % 2).wait()
            start(ca + 1, lane0, (ca + 1) % 2)
            compute(ca % 2)
        cb = NCH - 1
        x_copy(cb, lane0, cb % 2).wait()
        m_copy(cb, lane0, cb % 2).wait()
        compute(cb % 2)
        pltpu.sync_copy(acc, out_hbm.at[pl.ds(0, D), pl.ds(lane0, Q)])
        return 0

    lax.fori_loop(0, NQ, q_body, 0)


def _tc_body(m_ref, x_ref, o_ref):
    il = pl.program_id(1)
    m = m_ref[...].astype(jnp.float32)
    part = jnp.sum(x_ref[...] * m[:, None, :], axis=0)

    @pl.when(il == 0)
    def _():
        o_ref[...] = part

    @pl.when(il > 0)
    def _():
        o_ref[...] += part


_tc_call = pl.pallas_call(
    _tc_body,
    grid=(NBL, NLS),
    in_specs=[
        pl.BlockSpec((CLt, LB), lambda ib, il: (il, ib)),
        pl.BlockSpec((CLt, D, LB), lambda ib, il: (il, 0, ib)),
    ],
    out_specs=pl.BlockSpec((D, LB), lambda ib, il: (0, ib)),
    out_shape=jax.ShapeDtypeStruct((D, B), jnp.float32),
    compiler_params=pltpu.CompilerParams(
        dimension_semantics=("parallel", "arbitrary")),
)


def kernel(inputs, mask):
    x3 = jnp.transpose(inputs, (1, 2, 0))
    x2 = x3.reshape(L * D, B)
    mt = jnp.transpose(mask, (1, 0))
    m_sc = mt[L_TC:, :].astype(jnp.float32)
    out_sc = _agg_sc(x2, m_sc)
    out_tc = _tc_call(mt, x3)
    return jnp.transpose(out_sc + out_tc, (1, 0))
